# Initial kernel scaffold; baseline (speedup 1.0000x reference)
import jax
import jax.numpy as jnp
from jax.experimental import pallas as pl


def kernel(feature_map, anchor):
    raise NotImplementedError("write your pallas kernel here")


# speedup vs baseline: 1.0676x; 1.0676x over previous
import jax
import jax.numpy as jnp
from jax import lax
from jax.experimental import pallas as pl
from jax.experimental.pallas import tpu as pltpu
from jax.experimental.pallas import tpu_sc as plsc

_P = 7
_HALF = 16.0
_B, _C, _H, _W = 4, 96, 224, 224
_N = 512
_K = _N * _P * _P
_TOT = _B * _K
_NC, _NS = 2, 16
_NW = _NC * _NS
_PTS_W = _TOT // _NW
_CHUNK = 64
_NCH = _PTS_W // _CHUNK
_APW = _B * _N // _NW
_HW = _H * _W


def _sc_body(fm_hbm, anch_hbm, out_hbm, anch_v, idx_v, dx_v, dy_v, gbuf,
             obuf, gsem):
    wid = lax.axis_index("s") * _NC + lax.axis_index("c")
    pltpu.sync_copy(anch_hbm.at[pl.ds(wid * (_APW * 2), _APW * 2)], anch_v)
    hl = _HALF / _H
    scale = float(_H - 1)

    def chunk_body(cc, carry):
        kbase = wid * _PTS_W + cc * _CHUNK
        for v in range(_CHUNK // 16):
            k = kbase + 16 * v + lax.iota(jnp.int32, 16)
            g = lax.div(k, jnp.full((16,), _P * _P, jnp.int32))
            r = k - g * (_P * _P)
            i = lax.div(r, jnp.full((16,), _P, jnp.int32))
            j = r - i * _P
            b = lax.div(g, jnp.full((16,), _N, jnp.int32))
            nloc = g - wid * _APW
            ax = plsc.load_gather(anch_v, [2 * nloc])
            ay = plsc.load_gather(anch_v, [2 * nloc + 1])
            ti = i.astype(jnp.float32) * (1.0 / (_P - 1))
            tj = j.astype(jnp.float32) * (1.0 / (_P - 1))
            xmin = jnp.clip(ax - hl, 0.0, 1.0)
            xmax = jnp.clip(ax + hl, 0.0, 1.0)
            ymin = jnp.clip(ay - hl, 0.0, 1.0)
            ymax = jnp.clip(ay + hl, 0.0, 1.0)
            px = jnp.clip((xmin + (xmax - xmin) * ti) * scale, 0.0, scale)
            py = jnp.clip((ymin + (ymax - ymin) * tj) * scale, 0.0, scale)
            xb = jnp.minimum(px.astype(jnp.int32), _W - 2)
            yb = jnp.minimum(py.astype(jnp.int32), _H - 2)
            dx = px - xb.astype(jnp.float32)
            dy = py - yb.astype(jnp.float32)
            base = b * _HW + yb * _W + xb
            sl = pl.ds(16 * v, 16)
            idx_v[0, sl] = base
            idx_v[1, sl] = base + 1
            idx_v[2, sl] = base + _W
            idx_v[3, sl] = base + _W + 1
            dx_v[sl] = dx
            dy_v[sl] = dy
        cps = [pltpu.async_copy(fm_hbm.at[idx_v.at[c]], gbuf.at[c], gsem)
               for c in range(4)]
        for cp in cps:
            cp.wait()

        def pt_body(p, pc):
            dxp = dx_v[pl.ds(p, 16)][0]
            dyp = dy_v[pl.ds(p, 16)][0]
            for s in range(_C // 16):
                csl = pl.ds(16 * s, 16)
                vlt = gbuf[0, p, csl]
                vrt = gbuf[1, p, csl]
                vlb = gbuf[2, p, csl]
                vrb = gbuf[3, p, csl]
                vt = vlt + (vrt - vlt) * dxp
                vb = vlb + (vrb - vlb) * dxp
                obuf[p, csl] = vt + (vb - vt) * dyp
            return pc

        lax.fori_loop(0, _CHUNK, pt_body, 0)
        pltpu.sync_copy(obuf, out_hbm.at[pl.ds(kbase, _CHUNK)])
        return carry

    lax.fori_loop(0, _NCH, chunk_body, 0)


def kernel(feature_map, anchor):
    fm_rows = jnp.transpose(feature_map, (0, 2, 3, 1)).reshape(_B * _HW, _C)
    anch = anchor.reshape(_B * _N * 2)
    call = pl.kernel(
        _sc_body,
        out_type=jax.ShapeDtypeStruct((_TOT, _C), jnp.float32),
        mesh=plsc.VectorSubcoreMesh(core_axis_name="c", subcore_axis_name="s"),
        compiler_params=pltpu.CompilerParams(use_tc_tiling_on_sc=False, needs_layout_passes=False),
        scratch_types=[
            pltpu.VMEM((_APW * 2,), jnp.float32),
            pltpu.VMEM((4, _CHUNK), jnp.int32),
            pltpu.VMEM((_CHUNK + 16,), jnp.float32),
            pltpu.VMEM((_CHUNK + 16,), jnp.float32),
            pltpu.VMEM((4, _CHUNK, _C), jnp.float32),
            pltpu.VMEM((_CHUNK, _C), jnp.float32),
            pltpu.SemaphoreType.DMA,
        ],
    )
    out = call(fm_rows, anch)
    return out.reshape(_B, _K, _C)


# speedup vs baseline: 1.1851x; 1.1101x over previous
import jax
import jax.numpy as jnp
from jax import lax
from jax.experimental import pallas as pl
from jax.experimental.pallas import tpu as pltpu
from jax.experimental.pallas import tpu_sc as plsc

_P = 7
_HALF = 16.0
_B, _C, _H, _W = 4, 96, 224, 224
_N = 512
_K = _N * _P * _P
_TOT = _B * _K
_NC, _NS = 2, 16
_NW = _NC * _NS
_PTS_W = _TOT // _NW
_CHUNK = 64
_NCH = _PTS_W // _CHUNK
_APW = _B * _N // _NW
_HW = _H * _W
_CP = 128


def _sc_body(fm_hbm, anch_hbm, out_hbm, anch_v, idx_v, dx_v, dy_v, gbuf,
             obuf, gsem):
    wid = lax.axis_index("s") * _NC + lax.axis_index("c")
    pltpu.sync_copy(anch_hbm.at[pl.ds(wid * (_APW * 2), _APW * 2)], anch_v)
    hl = _HALF / _H
    scale = float(_H - 1)

    def chunk_body(cc, carry):
        kbase = wid * _PTS_W + cc * _CHUNK
        for v in range(_CHUNK // 16):
            k = kbase + 16 * v + lax.iota(jnp.int32, 16)
            g = lax.div(k, jnp.full((16,), _P * _P, jnp.int32))
            r = k - g * (_P * _P)
            i = lax.div(r, jnp.full((16,), _P, jnp.int32))
            j = r - i * _P
            b = lax.div(g, jnp.full((16,), _N, jnp.int32))
            nloc = g - wid * _APW
            ax = plsc.load_gather(anch_v, [2 * nloc])
            ay = plsc.load_gather(anch_v, [2 * nloc + 1])
            ti = i.astype(jnp.float32) * (1.0 / (_P - 1))
            tj = j.astype(jnp.float32) * (1.0 / (_P - 1))
            xmin = jnp.clip(ax - hl, 0.0, 1.0)
            xmax = jnp.clip(ax + hl, 0.0, 1.0)
            ymin = jnp.clip(ay - hl, 0.0, 1.0)
            ymax = jnp.clip(ay + hl, 0.0, 1.0)
            px = jnp.clip((xmin + (xmax - xmin) * ti) * scale, 0.0, scale)
            py = jnp.clip((ymin + (ymax - ymin) * tj) * scale, 0.0, scale)
            xb = jnp.minimum(px.astype(jnp.int32), _W - 2)
            yb = jnp.minimum(py.astype(jnp.int32), _H - 2)
            dx = px - xb.astype(jnp.float32)
            dy = py - yb.astype(jnp.float32)
            base = b * _HW + yb * _W + xb
            sl = pl.ds(16 * v, 16)
            idx_v[0, sl] = base
            idx_v[1, sl] = base + 1
            idx_v[2, sl] = base + _W
            idx_v[3, sl] = base + _W + 1
            dx_v[sl] = dx
            dy_v[sl] = dy
        cps = [pltpu.async_copy(fm_hbm.at[idx_v.at[c]], gbuf.at[c], gsem)
               for c in range(4)]
        for cp in cps:
            cp.wait()

        def pt_body(p, pc):
            dxp = dx_v[pl.ds(p, 16)][0]
            dyp = dy_v[pl.ds(p, 16)][0]
            for s in range(_C // 16):
                csl = pl.ds(16 * s, 16)
                vlt = gbuf[0, p, csl]
                vrt = gbuf[1, p, csl]
                vlb = gbuf[2, p, csl]
                vrb = gbuf[3, p, csl]
                vt = vlt + (vrt - vlt) * dxp
                vb = vlb + (vrb - vlb) * dxp
                obuf[p, csl] = vt + (vb - vt) * dyp
            return pc

        lax.fori_loop(0, _CHUNK, pt_body, 0)
        pltpu.sync_copy(obuf, out_hbm.at[pl.ds(kbase, _CHUNK)])
        return carry

    lax.fori_loop(0, _NCH, chunk_body, 0)


def kernel(feature_map, anchor):
    fm_rows = jnp.pad(jnp.transpose(feature_map, (0, 2, 3, 1)),
                      ((0, 0), (0, 0), (0, 0), (0, _CP - _C)))
    fm_rows = fm_rows.reshape(_B * _HW, _CP)
    anch = anchor.reshape(_B * _N * 2)
    call = pl.kernel(
        _sc_body,
        out_type=jax.ShapeDtypeStruct((_TOT, _CP), jnp.float32),
        mesh=plsc.VectorSubcoreMesh(core_axis_name="c", subcore_axis_name="s"),
        compiler_params=pltpu.CompilerParams(use_tc_tiling_on_sc=True, needs_layout_passes=False),
        scratch_types=[
            pltpu.VMEM((_APW * 2,), jnp.float32),
            pltpu.VMEM((4, _CHUNK), jnp.int32),
            pltpu.VMEM((_CHUNK + 16,), jnp.float32),
            pltpu.VMEM((_CHUNK + 16,), jnp.float32),
            pltpu.VMEM((4, _CHUNK, _CP), jnp.float32),
            pltpu.VMEM((_CHUNK, _CP), jnp.float32),
            pltpu.SemaphoreType.DMA,
        ],
    )
    out = call(fm_rows, anch)
    return out[:, :_C].reshape(_B, _K, _C)


# speedup vs baseline: 2.7920x; 2.3560x over previous
import jax
import jax.numpy as jnp
from jax import lax
from jax.experimental import pallas as pl
from jax.experimental.pallas import tpu as pltpu
from jax.experimental.pallas import tpu_sc as plsc

_P = 7
_HALF = 16.0
_B, _C, _H, _W = 4, 96, 224, 224
_N = 512
_K = _N * _P * _P
_TOT = _B * _K
_NC, _NS = 2, 16
_NW = _NC * _NS
_PTS_W = _TOT // _NW
_CHUNK = 64
_NCH = _PTS_W // _CHUNK
_APW = _B * _N // _NW
_HW = _H * _W
_CP = 128


_RB = 8


def _tc_relayout_body(fm_ref, out_ref):
    for hh in range(_RB):
        out_ref[pl.ds(_W * hh, _W), :_C] = fm_ref[0, :, hh, :].T


def _relayout(feature_map):
    return pl.pallas_call(
        _tc_relayout_body,
        grid=(_B, _H // _RB),
        in_specs=[pl.BlockSpec((1, _C, _RB, _W), lambda b, h: (b, 0, h, 0))],
        out_specs=pl.BlockSpec((_RB * _W, _CP),
                               lambda b, h: (b * (_H // _RB) + h, 0)),
        out_shape=jax.ShapeDtypeStruct((_B * _HW, _CP), jnp.float32),
    )(feature_map)


def _sc_body(fm_hbm, anch_hbm, out_hbm, anch_v, idx_v, dx_v, dy_v, gbuf,
             obuf, gsem0, gsem1):
    wid = lax.axis_index("s") * _NC + lax.axis_index("c")
    pltpu.sync_copy(anch_hbm.at[pl.ds(wid * (_APW * 2), _APW * 2)], anch_v)
    hl = _HALF / _H
    scale = float(_H - 1)
    gsems = (gsem0, gsem1)

    def compute_and_fire(cc, slot):
        kbase = wid * _PTS_W + cc * _CHUNK
        for v in range(_CHUNK // 16):
            k = kbase + 16 * v + lax.iota(jnp.int32, 16)
            g = lax.div(k, jnp.full((16,), _P * _P, jnp.int32))
            r = k - g * (_P * _P)
            i = lax.div(r, jnp.full((16,), _P, jnp.int32))
            j = r - i * _P
            b = lax.div(g, jnp.full((16,), _N, jnp.int32))
            nloc = g - wid * _APW
            ax = plsc.load_gather(anch_v, [2 * nloc])
            ay = plsc.load_gather(anch_v, [2 * nloc + 1])
            ti = i.astype(jnp.float32) * (1.0 / (_P - 1))
            tj = j.astype(jnp.float32) * (1.0 / (_P - 1))
            xmin = jnp.clip(ax - hl, 0.0, 1.0)
            xmax = jnp.clip(ax + hl, 0.0, 1.0)
            ymin = jnp.clip(ay - hl, 0.0, 1.0)
            ymax = jnp.clip(ay + hl, 0.0, 1.0)
            px = jnp.clip((xmin + (xmax - xmin) * ti) * scale, 0.0, scale)
            py = jnp.clip((ymin + (ymax - ymin) * tj) * scale, 0.0, scale)
            xb = jnp.minimum(px.astype(jnp.int32), _W - 2)
            yb = jnp.minimum(py.astype(jnp.int32), _H - 2)
            dx = px - xb.astype(jnp.float32)
            dy = py - yb.astype(jnp.float32)
            base = b * _HW + yb * _W + xb
            sl = pl.ds(16 * v, 16)
            idx_v[slot, 0, sl] = base
            idx_v[slot, 1, sl] = base + 1
            idx_v[slot, 2, sl] = base + _W
            idx_v[slot, 3, sl] = base + _W + 1
            dx_v[slot, sl] = dx
            dy_v[slot, sl] = dy
        for c in range(4):
            pltpu.async_copy(fm_hbm.at[idx_v.at[slot, c]],
                             gbuf.at[slot, c], gsems[slot])

    def drain(slot):
        for c in range(4):
            pltpu.make_async_copy(fm_hbm.at[idx_v.at[slot, c]],
                                  gbuf.at[slot, c], gsems[slot]).wait()

    def combine_and_store(cc, slot):
        drain(slot)
        kbase = wid * _PTS_W + cc * _CHUNK

        def pt_body(p, pc):
            dxp = dx_v[slot, pl.ds(p, 16)][0]
            dyp = dy_v[slot, pl.ds(p, 16)][0]
            for s in range(_C // 16):
                csl = pl.ds(16 * s, 16)
                vlt = gbuf[slot, 0, p, csl]
                vrt = gbuf[slot, 1, p, csl]
                vlb = gbuf[slot, 2, p, csl]
                vrb = gbuf[slot, 3, p, csl]
                vt = vlt + (vrt - vlt) * dxp
                vb = vlb + (vrb - vlb) * dxp
                obuf[slot, p, csl] = vt + (vb - vt) * dyp
            return pc

        lax.fori_loop(0, _CHUNK, pt_body, 0)
        pltpu.sync_copy(obuf.at[slot], out_hbm.at[pl.ds(kbase, _CHUNK)])

    compute_and_fire(0, 0)

    def pair_body(it, carry):
        cc = 2 * it
        compute_and_fire(cc + 1, 1)
        combine_and_store(cc, 0)
        compute_and_fire(cc + 2, 0)
        combine_and_store(cc + 1, 1)
        return carry

    lax.fori_loop(0, _NCH // 2, pair_body, 0)
    combine_and_store(_NCH - 1, 0)


def kernel(feature_map, anchor):
    fm_rows = _relayout(feature_map)
    anch = anchor.reshape(_B * _N * 2)
    call = pl.kernel(
        _sc_body,
        out_type=jax.ShapeDtypeStruct((_TOT, _CP), jnp.float32),
        mesh=plsc.VectorSubcoreMesh(core_axis_name="c", subcore_axis_name="s"),
        compiler_params=pltpu.CompilerParams(use_tc_tiling_on_sc=True,
                                             needs_layout_passes=False),
        scratch_types=[
            pltpu.VMEM((_APW * 2,), jnp.float32),
            pltpu.VMEM((2, 4, _CHUNK), jnp.int32),
            pltpu.VMEM((2, _CHUNK + 16), jnp.float32),
            pltpu.VMEM((2, _CHUNK + 16), jnp.float32),
            pltpu.VMEM((2, 4, _CHUNK, _CP), jnp.float32),
            pltpu.VMEM((2, _CHUNK, _CP), jnp.float32),
            pltpu.SemaphoreType.DMA,
            pltpu.SemaphoreType.DMA,
        ],
    )
    out = call(fm_rows, anch)
    return out[:, :_C].reshape(_B, _K, _C)
